# baseline (speedup 1.0000x reference)
import dataclasses

import jax
import jax.numpy as jnp
from jax import lax
from jax.experimental import pallas as pl
from jax.experimental.pallas import tpu as pltpu
from jax.experimental.pallas import tpu_sc as plsc

N = 10000
E = 320000
D = 128

NC = 2
NS = 16
NW = NC * NS
CH = 128

NP = 10240
PAD_ROWS = NP - N
RPW = NP // NS

NCHUNK = 80
KG = 16
NG = NCHUNK // KG
EPT = NCHUNK * CH
EP = EPT * NW
EROWS = EP // CH

_mesh = plsc.VectorSubcoreMesh(core_axis_name="c", subcore_axis_name="s")

_cp = pltpu.CompilerParams()
if "needs_layout_passes" in pltpu.CompilerParams.__dataclass_fields__:
    _cp = dataclasses.replace(_cp, needs_layout_passes=False)


def _hist_chunk(hist_v, dst_v, i):
    ones16 = jnp.ones((16,), jnp.float32)
    lanes = lax.iota(jnp.int32, 16)
    for k in range(CH // 16):
        idx16 = dst_v[i, pl.ds(16 * k, 16)]
        for j in range(16):
            plsc.addupdate_scatter(hist_v, [idx16], ones16, mask=lanes == j)


def _agg_loop(h_hbm, acc_sh, src_hbm, dst_hbm, wid, src_v, dst_v,
              rows0, rows1, sem0, sem1, ssem0, ssem1, hist_v=None):
    def wg(rows, sem, i):
        pltpu.make_async_copy(h_hbm.at[src_v.at[i]], rows, sem).wait()

    def ws(rows, ssem, i):
        pltpu.make_async_copy(rows, acc_sh.at[dst_v.at[i]], ssem).wait()

    @pl.loop(0, NG)
    def _(g):
        irows = pl.ds(wid * NCHUNK + g * KG, KG)
        pltpu.sync_copy(src_hbm.at[irows], src_v)
        pltpu.sync_copy(dst_hbm.at[irows], dst_v)
        pltpu.async_copy(h_hbm.at[src_v.at[0]], rows0, sem0)

        @pl.loop(0, KG, step=2)
        def _(i):
            wg(rows0, sem0, i)
            pltpu.async_copy(rows0, acc_sh.at[dst_v.at[i]], ssem0,
                             add=True)
            if hist_v is not None:
                _hist_chunk(hist_v, dst_v, i)

            @pl.when(i >= 1)
            def _():
                ws(rows1, ssem1, i)

            pltpu.async_copy(h_hbm.at[src_v.at[i + 1]], rows1, sem1)
            wg(rows1, sem1, i + 1)
            pltpu.async_copy(rows1, acc_sh.at[dst_v.at[i + 1]], ssem1,
                             add=True)
            if hist_v is not None:
                _hist_chunk(hist_v, dst_v, i + 1)

            @pl.when(i + 2 < KG)
            def _():
                ws(rows0, ssem0, i)
                pltpu.async_copy(h_hbm.at[src_v.at[i + 2]], rows0, sem0)

        ws(rows0, ssem0, 0)
        ws(rows1, ssem1, 0)


def _sc_agg_counts_body(h_hbm, src_hbm, dst_hbm, z_hbm,
                        out_hbm, cnt_hbm,
                        acc_sh, src_v, dst_v, rows0, rows1, hist_v,
                        sem0, sem1, ssem0, ssem1):
    c = lax.axis_index("c")
    s = lax.axis_index("s")
    wid = c * NS + s
    sl = pl.ds(s * RPW, RPW)
    osl = pl.ds(c * NP + s * RPW, RPW)

    pltpu.sync_copy(z_hbm, acc_sh.at[sl])

    @pl.loop(0, NP, step=16)
    def _(i):
        hist_v[pl.ds(i, 16)] = jnp.zeros((16,), jnp.float32)

    plsc.subcore_barrier()

    _agg_loop(h_hbm, acc_sh, src_hbm, dst_hbm, wid, src_v, dst_v,
              rows0, rows1, sem0, sem1, ssem0, ssem1, hist_v=hist_v)

    plsc.subcore_barrier()
    pltpu.sync_copy(acc_sh.at[sl], out_hbm.at[osl])
    pltpu.sync_copy(hist_v, cnt_hbm.at[pl.ds(wid * NP, NP)])


def _sc_agg_body(h_hbm, src_hbm, dst_hbm, z_hbm,
                 out_hbm,
                 acc_sh, src_v, dst_v, rows0, rows1,
                 sem0, sem1, ssem0, ssem1):
    c = lax.axis_index("c")
    s = lax.axis_index("s")
    wid = c * NS + s
    sl = pl.ds(s * RPW, RPW)

    pltpu.sync_copy(z_hbm, acc_sh.at[sl])
    plsc.subcore_barrier()

    _agg_loop(h_hbm, acc_sh, src_hbm, dst_hbm, wid, src_v, dst_v,
              rows0, rows1, sem0, sem1, ssem0, ssem1)

    plsc.subcore_barrier()
    pltpu.sync_copy(acc_sh.at[sl], out_hbm.at[pl.ds(c * NP + s * RPW, RPW)])


_sc_agg_counts = pl.kernel(
    _sc_agg_counts_body,
    out_type=(jax.ShapeDtypeStruct((NC * NP, D), jnp.float32),
              jax.ShapeDtypeStruct((NW * NP,), jnp.float32)),
    mesh=_mesh,
    scratch_types=[
        pltpu.VMEM_SHARED((NP, D), jnp.float32),
        pltpu.VMEM((KG, CH), jnp.int32),
        pltpu.VMEM((KG, CH), jnp.int32),
        pltpu.VMEM((CH, D), jnp.float32),
        pltpu.VMEM((CH, D), jnp.float32),
        pltpu.VMEM((NP,), jnp.float32),
        pltpu.SemaphoreType.DMA,
        pltpu.SemaphoreType.DMA,
        pltpu.SemaphoreType.DMA,
        pltpu.SemaphoreType.DMA,
    ],
    compiler_params=_cp,
)

_sc_agg = pl.kernel(
    _sc_agg_body,
    out_type=jax.ShapeDtypeStruct((NC * NP, D), jnp.float32),
    mesh=_mesh,
    scratch_types=[
        pltpu.VMEM_SHARED((NP, D), jnp.float32),
        pltpu.VMEM((KG, CH), jnp.int32),
        pltpu.VMEM((KG, CH), jnp.int32),
        pltpu.VMEM((CH, D), jnp.float32),
        pltpu.VMEM((CH, D), jnp.float32),
        pltpu.SemaphoreType.DMA,
        pltpu.SemaphoreType.DMA,
        pltpu.SemaphoreType.DMA,
        pltpu.SemaphoreType.DMA,
    ],
)

BM = 2000


def _tc_right_body(h_ref, wr_ref, b_ref, o_ref):
    o_ref[...] = (jnp.dot(h_ref[...], wr_ref[...],
                          preferred_element_type=jnp.float32) + b_ref[...])


def _tc_mid_body(acc_ref, cnt_ref, r_ref, wl_ref, wr2_ref, b2_ref,
                 h_ref, r2_ref):
    s = acc_ref[0] + acc_ref[1]
    cnt = jnp.sum(cnt_ref[...], axis=1)
    agg = s * (1.0 / jnp.maximum(cnt, 1.0))[:, None]
    h1 = jnp.maximum(
        jnp.dot(agg, wl_ref[...], preferred_element_type=jnp.float32)
        + r_ref[...], 0.0)
    h_ref[...] = h1
    r2_ref[...] = (jnp.dot(h1, wr2_ref[...],
                           preferred_element_type=jnp.float32) + b2_ref[...])


def _tc_final_body(acc_ref, cnt_ref, r_ref, wl_ref, w3_ref, b3_ref, o_ref):
    s = acc_ref[0] + acc_ref[1]
    cnt = jnp.sum(cnt_ref[...], axis=1)
    agg = s * (1.0 / jnp.maximum(cnt, 1.0))[:, None]
    h2 = jnp.maximum(
        jnp.dot(agg, wl_ref[...], preferred_element_type=jnp.float32)
        + r_ref[...], 0.0)
    o_ref[...] = jnp.maximum(
        jnp.dot(h2, w3_ref[...], preferred_element_type=jnp.float32)
        + b3_ref[...], 0.0)


_spec_acc = pl.BlockSpec((NC, BM, D), lambda i: (0, i, 0))
_spec_cnt = pl.BlockSpec((BM, NW), lambda i: (i, 0))
_spec_h = pl.BlockSpec((BM, D), lambda i: (i, 0))
_spec_w = pl.BlockSpec((D, D), lambda i: (0, 0))
_spec_b = pl.BlockSpec((1, D), lambda i: (0, 0))
_out_h = jax.ShapeDtypeStruct((N, D), jnp.float32)

_tc_right = pl.pallas_call(
    _tc_right_body,
    grid=(N // BM,),
    in_specs=[_spec_h, _spec_w, _spec_b],
    out_specs=_spec_h,
    out_shape=_out_h,
)

_tc_mid = pl.pallas_call(
    _tc_mid_body,
    grid=(N // BM,),
    in_specs=[_spec_acc, _spec_cnt, _spec_h, _spec_w, _spec_w, _spec_b],
    out_specs=(_spec_h, _spec_h),
    out_shape=(_out_h, _out_h),
)

_tc_final = pl.pallas_call(
    _tc_final_body,
    grid=(N // BM,),
    in_specs=[_spec_acc, _spec_cnt, _spec_h, _spec_w, _spec_w, _spec_b],
    out_specs=_spec_h,
    out_shape=_out_h,
)


def kernel(x, edge_index, W1l, b1l, W1r, W2l, b2l, W2r, W3, b3):
    src = edge_index[0].astype(jnp.int32)
    dst = edge_index[1].astype(jnp.int32)
    pidx = jnp.arange(EP - E, dtype=jnp.int32)
    src_p = jnp.concatenate([src, pidx % N]).reshape(EROWS, CH)
    dst_p = jnp.concatenate([dst, N + pidx % PAD_ROWS]).reshape(EROWS, CH)

    z_rows = jnp.zeros((RPW, D), jnp.float32)

    acc1, cnt = _sc_agg_counts(x, src_p, dst_p, z_rows)
    r1 = _tc_right(x, W1r, b1l.reshape(1, D))
    acc1 = acc1.reshape(NC, NP, D)
    cnt = cnt.reshape(NW, NP).T
    h1, r2 = _tc_mid(acc1, cnt, r1, W1l, W2r, b2l.reshape(1, D))
    acc2 = _sc_agg(h1, src_p, dst_p, z_rows).reshape(NC, NP, D)
    out = _tc_final(acc2, cnt, r2, W2l, W3, b3.reshape(1, D))
    return out


# speedup vs baseline: 1.0132x; 1.0132x over previous
import dataclasses

import jax
import jax.numpy as jnp
from jax import lax
from jax.experimental import pallas as pl
from jax.experimental.pallas import tpu as pltpu
from jax.experimental.pallas import tpu_sc as plsc

N = 10000
E = 320000
D = 128

NC = 2
NS = 16
NW = NC * NS
CH = 128

NP = 10240
PAD_ROWS = NP - N
RPW = NP // NS

NCHUNK = 80
KG = 16
NG = NCHUNK // KG
EPT = NCHUNK * CH
EP = EPT * NW
EROWS = EP // CH

_mesh = plsc.VectorSubcoreMesh(core_axis_name="c", subcore_axis_name="s")

_cp = pltpu.CompilerParams()
if "needs_layout_passes" in pltpu.CompilerParams.__dataclass_fields__:
    _cp = dataclasses.replace(_cp, needs_layout_passes=False)


def _hist_chunk(hist_v, dst_v, i):
    ones16 = jnp.ones((16,), jnp.float32)
    lanes = lax.iota(jnp.int32, 16)
    for k in range(CH // 16):
        idx16 = dst_v[i, pl.ds(16 * k, 16)]
        for j in range(16):
            plsc.addupdate_scatter(hist_v, [idx16], ones16, mask=lanes == j)


def _agg_loop(h_hbm, acc_sh, src_hbm, dst_hbm, wid, src_v, dst_v,
              rows0, rows1, sem0, sem1, hist_v=None):
    @pl.loop(0, NG)
    def _(g):
        irows = pl.ds(wid * NCHUNK + g * KG, KG)
        pltpu.sync_copy(src_hbm.at[irows], src_v)
        pltpu.sync_copy(dst_hbm.at[irows], dst_v)
        pltpu.async_copy(h_hbm.at[src_v.at[0]], rows0, sem0)

        @pl.loop(0, KG, step=2)
        def _(i):
            pltpu.make_async_copy(h_hbm.at[src_v.at[i]], rows0, sem0).wait()
            pltpu.async_copy(h_hbm.at[src_v.at[i + 1]], rows1, sem1)
            pltpu.sync_copy(rows0, acc_sh.at[dst_v.at[i]], add=True)
            if hist_v is not None:
                _hist_chunk(hist_v, dst_v, i)
            pltpu.make_async_copy(h_hbm.at[src_v.at[i + 1]], rows1, sem1).wait()

            @pl.when(i + 2 < KG)
            def _():
                pltpu.async_copy(h_hbm.at[src_v.at[i + 2]], rows0, sem0)

            pltpu.sync_copy(rows1, acc_sh.at[dst_v.at[i + 1]], add=True)
            if hist_v is not None:
                _hist_chunk(hist_v, dst_v, i + 1)


def _sc_agg_counts_body(h_hbm, src_hbm, dst_hbm, z_hbm,
                        out_hbm, cnt_hbm,
                        acc_sh, src_v, dst_v, rows0, rows1, hist_v,
                        sem0, sem1):
    c = lax.axis_index("c")
    s = lax.axis_index("s")
    wid = c * NS + s
    sl = pl.ds(s * RPW, RPW)
    osl = pl.ds(c * NP + s * RPW, RPW)

    pltpu.sync_copy(z_hbm, acc_sh.at[sl])

    @pl.loop(0, NP, step=16)
    def _(i):
        hist_v[pl.ds(i, 16)] = jnp.zeros((16,), jnp.float32)

    plsc.subcore_barrier()

    _agg_loop(h_hbm, acc_sh, src_hbm, dst_hbm, wid, src_v, dst_v,
              rows0, rows1, sem0, sem1, hist_v=hist_v)

    plsc.subcore_barrier()
    pltpu.sync_copy(acc_sh.at[sl], out_hbm.at[osl])
    pltpu.sync_copy(hist_v, cnt_hbm.at[pl.ds(wid * NP, NP)])


def _sc_agg_body(h_hbm, src_hbm, dst_hbm, z_hbm,
                 out_hbm,
                 acc_sh, src_v, dst_v, rows0, rows1,
                 sem0, sem1):
    c = lax.axis_index("c")
    s = lax.axis_index("s")
    wid = c * NS + s
    sl = pl.ds(s * RPW, RPW)

    pltpu.sync_copy(z_hbm, acc_sh.at[sl])
    plsc.subcore_barrier()

    _agg_loop(h_hbm, acc_sh, src_hbm, dst_hbm, wid, src_v, dst_v,
              rows0, rows1, sem0, sem1)

    plsc.subcore_barrier()
    pltpu.sync_copy(acc_sh.at[sl], out_hbm.at[pl.ds(c * NP + s * RPW, RPW)])


_sc_agg_counts = pl.kernel(
    _sc_agg_counts_body,
    out_type=(jax.ShapeDtypeStruct((NC * NP, D), jnp.float32),
              jax.ShapeDtypeStruct((NW * NP,), jnp.float32)),
    mesh=_mesh,
    scratch_types=[
        pltpu.VMEM_SHARED((NP, D), jnp.float32),
        pltpu.VMEM((KG, CH), jnp.int32),
        pltpu.VMEM((KG, CH), jnp.int32),
        pltpu.VMEM((CH, D), jnp.float32),
        pltpu.VMEM((CH, D), jnp.float32),
        pltpu.VMEM((NP,), jnp.float32),
        pltpu.SemaphoreType.DMA,
        pltpu.SemaphoreType.DMA,
    ],
    compiler_params=_cp,
)

_sc_agg = pl.kernel(
    _sc_agg_body,
    out_type=jax.ShapeDtypeStruct((NC * NP, D), jnp.float32),
    mesh=_mesh,
    scratch_types=[
        pltpu.VMEM_SHARED((NP, D), jnp.float32),
        pltpu.VMEM((KG, CH), jnp.int32),
        pltpu.VMEM((KG, CH), jnp.int32),
        pltpu.VMEM((CH, D), jnp.float32),
        pltpu.VMEM((CH, D), jnp.float32),
        pltpu.SemaphoreType.DMA,
        pltpu.SemaphoreType.DMA,
    ],
)

BM = 2000


def _tc_right_body(h_ref, wr_ref, b_ref, o_ref):
    o_ref[...] = (jnp.dot(h_ref[...], wr_ref[...],
                          preferred_element_type=jnp.float32) + b_ref[...])


def _tc_mid_body(acc_ref, cnt_ref, r_ref, wl_ref, wr2_ref, b2_ref,
                 h_ref, r2_ref):
    s = acc_ref[0] + acc_ref[1]
    cnt = jnp.sum(cnt_ref[...], axis=1)
    agg = s * (1.0 / jnp.maximum(cnt, 1.0))[:, None]
    h1 = jnp.maximum(
        jnp.dot(agg, wl_ref[...], preferred_element_type=jnp.float32)
        + r_ref[...], 0.0)
    h_ref[...] = h1
    r2_ref[...] = (jnp.dot(h1, wr2_ref[...],
                           preferred_element_type=jnp.float32) + b2_ref[...])


def _tc_final_body(acc_ref, cnt_ref, r_ref, wl_ref, w3_ref, b3_ref, o_ref):
    s = acc_ref[0] + acc_ref[1]
    cnt = jnp.sum(cnt_ref[...], axis=1)
    agg = s * (1.0 / jnp.maximum(cnt, 1.0))[:, None]
    h2 = jnp.maximum(
        jnp.dot(agg, wl_ref[...], preferred_element_type=jnp.float32)
        + r_ref[...], 0.0)
    o_ref[...] = jnp.maximum(
        jnp.dot(h2, w3_ref[...], preferred_element_type=jnp.float32)
        + b3_ref[...], 0.0)


_spec_acc = pl.BlockSpec((NC, BM, D), lambda i: (0, i, 0))
_spec_cnt = pl.BlockSpec((BM, NW), lambda i: (i, 0))
_spec_h = pl.BlockSpec((BM, D), lambda i: (i, 0))
_spec_w = pl.BlockSpec((D, D), lambda i: (0, 0))
_spec_b = pl.BlockSpec((1, D), lambda i: (0, 0))
_out_h = jax.ShapeDtypeStruct((N, D), jnp.float32)

_tc_right = pl.pallas_call(
    _tc_right_body,
    grid=(N // BM,),
    in_specs=[_spec_h, _spec_w, _spec_b],
    out_specs=_spec_h,
    out_shape=_out_h,
)

_tc_mid = pl.pallas_call(
    _tc_mid_body,
    grid=(N // BM,),
    in_specs=[_spec_acc, _spec_cnt, _spec_h, _spec_w, _spec_w, _spec_b],
    out_specs=(_spec_h, _spec_h),
    out_shape=(_out_h, _out_h),
)

_tc_final = pl.pallas_call(
    _tc_final_body,
    grid=(N // BM,),
    in_specs=[_spec_acc, _spec_cnt, _spec_h, _spec_w, _spec_w, _spec_b],
    out_specs=_spec_h,
    out_shape=_out_h,
)


def kernel(x, edge_index, W1l, b1l, W1r, W2l, b2l, W2r, W3, b3):
    src = edge_index[0].astype(jnp.int32)
    dst = edge_index[1].astype(jnp.int32)
    pidx = jnp.arange(EP - E, dtype=jnp.int32)
    src_p = jnp.concatenate([src, pidx % N]).reshape(EROWS, CH)
    dst_p = jnp.concatenate([dst, N + pidx % PAD_ROWS]).reshape(EROWS, CH)

    z_rows = jnp.zeros((RPW, D), jnp.float32)

    acc1, cnt = _sc_agg_counts(x, src_p, dst_p, z_rows)
    r1 = _tc_right(x, W1r, b1l.reshape(1, D))
    acc1 = acc1.reshape(NC, NP, D)
    cnt = cnt.reshape(NW, NP).T
    h1, r2 = _tc_mid(acc1, cnt, r1, W1l, W2r, b2l.reshape(1, D))
    acc2 = _sc_agg(h1, src_p, dst_p, z_rows).reshape(NC, NP, D)
    out = _tc_final(acc2, cnt, r2, W2l, W3, b3.reshape(1, D))
    return out


# speedup vs baseline: 1.0241x; 1.0108x over previous
import dataclasses

import jax
import jax.numpy as jnp
import numpy as np
from jax import lax
from jax.experimental import pallas as pl
from jax.experimental.pallas import tpu as pltpu
from jax.experimental.pallas import tpu_sc as plsc

N = 10000
E = 320000
D = 128

NC = 2
NS = 16
NW = NC * NS
CH = 128

NP = 10240
PAD_ROWS = NP - N
RPW = NP // NS

NCHUNK = 80
KG = 16
NG = NCHUNK // KG
EPT = NCHUNK * CH
EP = EPT * NW
EROWS = EP // CH

_mesh = plsc.VectorSubcoreMesh(core_axis_name="c", subcore_axis_name="s")

_cp = pltpu.CompilerParams()
if "needs_layout_passes" in pltpu.CompilerParams.__dataclass_fields__:
    _cp = dataclasses.replace(_cp, needs_layout_passes=False)


def _hist_chunk(hist_v, dst_v, i):
    ones16 = jnp.ones((16,), jnp.float32)
    lanes = lax.iota(jnp.int32, 16)
    for k in range(CH // 16):
        idx16 = dst_v[i, pl.ds(16 * k, 16)]
        for j in range(16):
            plsc.addupdate_scatter(hist_v, [idx16], ones16, mask=lanes == j)


def _agg_loop(h_hbm, acc_sh, src_hbm, dst_hbm, wid, src_v, dst_v,
              rows0, rows1, sem0, sem1, hist_v=None):
    @pl.loop(0, NG)
    def _(g):
        irows = pl.ds(wid * NCHUNK + g * KG, KG)
        pltpu.sync_copy(src_hbm.at[irows], src_v)
        pltpu.sync_copy(dst_hbm.at[irows], dst_v)
        pltpu.async_copy(h_hbm.at[src_v.at[0]], rows0, sem0)

        @pl.loop(0, KG, step=2)
        def _(i):
            pltpu.make_async_copy(h_hbm.at[src_v.at[i]], rows0, sem0).wait()
            pltpu.async_copy(h_hbm.at[src_v.at[i + 1]], rows1, sem1)
            pltpu.sync_copy(rows0, acc_sh.at[dst_v.at[i]], add=True)
            if hist_v is not None:
                _hist_chunk(hist_v, dst_v, i)
            pltpu.make_async_copy(h_hbm.at[src_v.at[i + 1]], rows1, sem1).wait()

            @pl.when(i + 2 < KG)
            def _():
                pltpu.async_copy(h_hbm.at[src_v.at[i + 2]], rows0, sem0)

            pltpu.sync_copy(rows1, acc_sh.at[dst_v.at[i + 1]], add=True)
            if hist_v is not None:
                _hist_chunk(hist_v, dst_v, i + 1)


def _sc_agg_counts_body(h_hbm, src_hbm, dst_hbm, z_hbm,
                        out_hbm, cnt_hbm,
                        acc_sh, src_v, dst_v, rows0, rows1, hist_v,
                        sem0, sem1):
    c = lax.axis_index("c")
    s = lax.axis_index("s")
    wid = c * NS + s
    sl = pl.ds(s * RPW, RPW)
    osl = pl.ds(c * NP + s * RPW, RPW)

    pltpu.sync_copy(z_hbm, acc_sh.at[sl])

    @pl.loop(0, NP, step=16)
    def _(i):
        hist_v[pl.ds(i, 16)] = jnp.zeros((16,), jnp.float32)

    plsc.subcore_barrier()

    _agg_loop(h_hbm, acc_sh, src_hbm, dst_hbm, wid, src_v, dst_v,
              rows0, rows1, sem0, sem1, hist_v=hist_v)

    plsc.subcore_barrier()
    pltpu.sync_copy(acc_sh.at[sl], out_hbm.at[osl])
    pltpu.sync_copy(hist_v, cnt_hbm.at[pl.ds(wid * NP, NP)])


def _sc_agg_body(h_hbm, src_hbm, dst_hbm, z_hbm,
                 out_hbm,
                 acc_sh, src_v, dst_v, rows0, rows1,
                 sem0, sem1):
    c = lax.axis_index("c")
    s = lax.axis_index("s")
    wid = c * NS + s
    sl = pl.ds(s * RPW, RPW)

    pltpu.sync_copy(z_hbm, acc_sh.at[sl])
    plsc.subcore_barrier()

    _agg_loop(h_hbm, acc_sh, src_hbm, dst_hbm, wid, src_v, dst_v,
              rows0, rows1, sem0, sem1)

    plsc.subcore_barrier()
    pltpu.sync_copy(acc_sh.at[sl], out_hbm.at[pl.ds(c * NP + s * RPW, RPW)])


_sc_agg_counts = pl.kernel(
    _sc_agg_counts_body,
    out_type=(jax.ShapeDtypeStruct((NC * NP, D), jnp.float32),
              jax.ShapeDtypeStruct((NW * NP,), jnp.float32)),
    mesh=_mesh,
    scratch_types=[
        pltpu.VMEM_SHARED((NP, D), jnp.float32),
        pltpu.VMEM((KG, CH), jnp.int32),
        pltpu.VMEM((KG, CH), jnp.int32),
        pltpu.VMEM((CH, D), jnp.float32),
        pltpu.VMEM((CH, D), jnp.float32),
        pltpu.VMEM((NP,), jnp.float32),
        pltpu.SemaphoreType.DMA,
        pltpu.SemaphoreType.DMA,
    ],
    compiler_params=_cp,
)

_sc_agg = pl.kernel(
    _sc_agg_body,
    out_type=jax.ShapeDtypeStruct((NC * NP, D), jnp.float32),
    mesh=_mesh,
    scratch_types=[
        pltpu.VMEM_SHARED((NP, D), jnp.float32),
        pltpu.VMEM((KG, CH), jnp.int32),
        pltpu.VMEM((KG, CH), jnp.int32),
        pltpu.VMEM((CH, D), jnp.float32),
        pltpu.VMEM((CH, D), jnp.float32),
        pltpu.SemaphoreType.DMA,
        pltpu.SemaphoreType.DMA,
    ],
)

BM = 2000


def _tc_right_body(h_ref, wr_ref, b_ref, o_ref):
    o_ref[...] = (jnp.dot(h_ref[...], wr_ref[...],
                          preferred_element_type=jnp.float32) + b_ref[...])


def _tc_mid_body(acc_ref, cnt_ref, r_ref, wl_ref, wr2_ref, b2_ref,
                 h_ref, r2_ref):
    s = acc_ref[0] + acc_ref[1]
    cnt = jnp.sum(cnt_ref[...], axis=1)
    agg = s * (1.0 / jnp.maximum(cnt, 1.0))[:, None]
    h1 = jnp.maximum(
        jnp.dot(agg, wl_ref[...], preferred_element_type=jnp.float32)
        + r_ref[...], 0.0)
    h_ref[...] = h1
    r2_ref[...] = (jnp.dot(h1, wr2_ref[...],
                           preferred_element_type=jnp.float32) + b2_ref[...])


def _tc_final_body(acc_ref, cnt_ref, r_ref, wl_ref, w3_ref, b3_ref, o_ref):
    s = acc_ref[0] + acc_ref[1]
    cnt = jnp.sum(cnt_ref[...], axis=1)
    agg = s * (1.0 / jnp.maximum(cnt, 1.0))[:, None]
    h2 = jnp.maximum(
        jnp.dot(agg, wl_ref[...], preferred_element_type=jnp.float32)
        + r_ref[...], 0.0)
    o_ref[...] = jnp.maximum(
        jnp.dot(h2, w3_ref[...], preferred_element_type=jnp.float32)
        + b3_ref[...], 0.0)


_spec_acc = pl.BlockSpec((NC, BM, D), lambda i: (0, i, 0))
_spec_cnt = pl.BlockSpec((BM, NW), lambda i: (i, 0))
_spec_h = pl.BlockSpec((BM, D), lambda i: (i, 0))
_spec_w = pl.BlockSpec((D, D), lambda i: (0, 0))
_spec_b = pl.BlockSpec((1, D), lambda i: (0, 0))
_out_h = jax.ShapeDtypeStruct((N, D), jnp.float32)

_tc_right = pl.pallas_call(
    _tc_right_body,
    grid=(N // BM,),
    in_specs=[_spec_h, _spec_w, _spec_b],
    out_specs=_spec_h,
    out_shape=_out_h,
)

_tc_mid = pl.pallas_call(
    _tc_mid_body,
    grid=(N // BM,),
    in_specs=[_spec_acc, _spec_cnt, _spec_h, _spec_w, _spec_w, _spec_b],
    out_specs=(_spec_h, _spec_h),
    out_shape=(_out_h, _out_h),
)

_tc_final = pl.pallas_call(
    _tc_final_body,
    grid=(N // BM,),
    in_specs=[_spec_acc, _spec_cnt, _spec_h, _spec_w, _spec_w, _spec_b],
    out_specs=_spec_h,
    out_shape=_out_h,
)


def kernel(x, edge_index, W1l, b1l, W1r, W2l, b2l, W2r, W3, b3):
    src = edge_index[0].astype(jnp.int32).reshape(E // CH, CH)
    dst = edge_index[1].astype(jnp.int32).reshape(E // CH, CH)
    pidx = np.arange(EP - E, dtype=np.int32)
    pad_src = jnp.asarray((pidx % N).reshape(-1, CH))
    pad_dst = jnp.asarray((N + pidx % PAD_ROWS).reshape(-1, CH))
    src_p = jnp.concatenate([src, pad_src], axis=0)
    dst_p = jnp.concatenate([dst, pad_dst], axis=0)

    z_rows = jnp.zeros((RPW, D), jnp.float32)

    acc1, cnt = _sc_agg_counts(x, src_p, dst_p, z_rows)
    r1 = _tc_right(x, W1r, b1l.reshape(1, D))
    acc1 = acc1.reshape(NC, NP, D)
    cnt = cnt.reshape(NW, NP).T
    h1, r2 = _tc_mid(acc1, cnt, r1, W1l, W2r, b2l.reshape(1, D))
    acc2 = _sc_agg(h1, src_p, dst_p, z_rows).reshape(NC, NP, D)
    out = _tc_final(acc2, cnt, r2, W2l, W3, b3.reshape(1, D))
    return out
